# baseline (speedup 1.0000x reference)
import jax
import jax.numpy as jnp
from jax import lax
from jax.experimental import pallas as pl
from jax.experimental.pallas import tpu as pltpu
from jax.experimental.pallas import tpu_sc as plsc

_FPS = 25.0
_LOSS_SCALE = 1e-07
_B, _N, _S = 16, 2048, 512
_CN = _N + 16
_L = 16
_SCH = 16


_GR = 16
_RPS = _B * _N // _GR


def _argmax_body(hx_ref, hy_ref, out_ref):
    iot = lax.broadcasted_iota(jnp.int32, (_RPS, 128), 1)

    def amax(z):
        v = z[:, 0:128]
        fi = iot
        for cc in (1, 2, 3):
            zc = z[:, cc * 128:(cc + 1) * 128]
            upd = zc > v
            v = jnp.where(upd, zc, v)
            fi = jnp.where(upd, iot + cc * 128, fi)
        mx = jnp.max(v, axis=-1, keepdims=True)
        cand = jnp.where(v == mx, fi, _S)
        return jnp.min(cand, axis=-1).astype(jnp.float32)

    out_ref[0, 0, :] = amax(hx_ref[0])
    out_ref[0, 1, :] = amax(hy_ref[0])


_argmax_call = pl.pallas_call(
    _argmax_body,
    grid=(_GR,),
    in_specs=[
        pl.BlockSpec((1, _RPS, _S), lambda i: (i, 0, 0)),
        pl.BlockSpec((1, _RPS, _S), lambda i: (i, 0, 0)),
    ],
    out_specs=pl.BlockSpec((1, 2, _RPS), lambda i: (i, 0, 0)),
    out_shape=jax.ShapeDtypeStruct((_GR, 2, _RPS), jnp.float32),
)


_GATHER_DNUMS = lax.GatherDimensionNumbers(
    offset_dims=(), collapsed_slice_dims=(0,), start_index_map=(0,))


def _gather16(x, idx):
    return lax.gather(x, idx[:, None], _GATHER_DNUMS, (1,),
                      mode=lax.GatherScatterMode.PROMISE_IN_BOUNDS)


def _sc1_labels_body(lx_hbm, ly_hbm, meta_hbm, dens_hbm,
                     lxv, lyv, cpfv, clxv, clyv, mfv, srcv, kbv, densv):
    c = lax.axis_index("c")
    s = lax.axis_index("s")
    wid = s * 2 + c

    @pl.when(wid < _B)
    def _():
        b = wid
        pltpu.sync_copy(lx_hbm.at[b], lxv)
        pltpu.sync_copy(ly_hbm.at[b], lyv)
        iota = lax.iota(jnp.int32, _L)
        zf = jnp.zeros((_L,), jnp.float32)
        zi = jnp.zeros((_L,), jnp.int32)
        l15 = jnp.full((_L,), _L - 1, jnp.int32)

        def sup1(sc_i, carry):
            kofs_s, kof_v, den1, den2 = carry
            kbuf = zf
            for t in range(_SCH):
                off = sc_i * (_SCH * _L) + t * _L
                lx = lxv[pl.ds(off, _L)]
                ly = lyv[pl.ds(off, _L)]
                m = jnp.logical_and(lx != 0.0, ly != 0.0)
                mi = jnp.where(m, 1, 0)
                mf = jnp.where(m, 1.0, 0.0)
                incl = mi
                for d in (1, 2, 4, 8):
                    sh = _gather16(incl, jnp.maximum(iota - d, 0))
                    incl = incl + jnp.where(iota >= d, sh, 0)
                src = zi
                for w in (8, 4, 2, 1):
                    probe = _gather16(incl, src + (w - 1))
                    src = src + jnp.where(probe < iota + 1, w, 0)
                mfv[pl.ds(off, _L)] = mf
                srcv[pl.ds(off, _L)] = src.astype(jnp.float32)
                cpfv[pl.ds(kofs_s, _L)] = (off + src).astype(jnp.float32)
                clxv[pl.ds(kofs_s, _L)] = _gather16(lx, src)
                clyv[pl.ds(kofs_s, _L)] = _gather16(ly, src)
                kbuf = jnp.where(iota == t, kof_v.astype(jnp.float32), kbuf)
                pos = off + iota
                den1 = den1 + jnp.where(pos >= 1, mf, 0.0)
                den2 = den2 + jnp.where(pos >= 2, mf, 0.0)
                kofs_s = kofs_s + jnp.squeeze(lax.slice(incl, (_L - 1,), (_L,)))
                kof_v = kof_v + _gather16(incl, l15)
            kbv[pl.ds(sc_i * _SCH, _L)] = kbuf
            return kofs_s, kof_v, den1, den2

        _, kof_v, den1, den2 = lax.fori_loop(
            0, _N // _L // _SCH, sup1, (jnp.int32(0), zi, zf, zf))
        kbv[pl.ds(_N // _L, _L)] = kof_v.astype(jnp.float32)
        densv[0, :] = den1
        densv[1, :] = den2
        pltpu.sync_copy(cpfv, meta_hbm.at[b, 0])
        pltpu.sync_copy(clxv, meta_hbm.at[b, 1])
        pltpu.sync_copy(clyv, meta_hbm.at[b, 2])
        pltpu.sync_copy(mfv, meta_hbm.at[b, 3])
        pltpu.sync_copy(srcv, meta_hbm.at[b, 4])
        pltpu.sync_copy(kbv, meta_hbm.at[b, 5])
        pltpu.sync_copy(densv, dens_hbm.at[b])


_sc1_call = pl.kernel(
    _sc1_labels_body,
    out_type=(jax.ShapeDtypeStruct((_B, 6, _CN), jnp.float32),
              jax.ShapeDtypeStruct((_B, 2, _L), jnp.float32)),
    mesh=plsc.VectorSubcoreMesh(core_axis_name="c", subcore_axis_name="s"),
    scratch_types=[
        pltpu.VMEM((_N,), jnp.float32),
        pltpu.VMEM((_N,), jnp.float32),
        pltpu.VMEM((_CN,), jnp.float32),
        pltpu.VMEM((_CN,), jnp.float32),
        pltpu.VMEM((_CN,), jnp.float32),
        pltpu.VMEM((_CN,), jnp.float32),
        pltpu.VMEM((_CN,), jnp.float32),
        pltpu.VMEM((_CN,), jnp.float32),
        pltpu.VMEM((2, _L), jnp.float32),
    ],
)


def _sc2_loss_body(px_hbm, py_hbm, meta_hbm, out_hbm,
                   pxv, pyv, cpfv, clxv, clyv, mfv, srcv, kbv, exv, eyv, resv):
    c = lax.axis_index("c")
    s = lax.axis_index("s")
    wid = s * 2 + c

    @pl.when(wid < _B)
    def _():
        b = wid
        pltpu.sync_copy(px_hbm.at[b], pxv)
        pltpu.sync_copy(py_hbm.at[b], pyv)
        pltpu.sync_copy(meta_hbm.at[b, 0], cpfv)
        pltpu.sync_copy(meta_hbm.at[b, 1], clxv)
        pltpu.sync_copy(meta_hbm.at[b, 2], clyv)
        pltpu.sync_copy(meta_hbm.at[b, 3], mfv)
        pltpu.sync_copy(meta_hbm.at[b, 4], srcv)
        pltpu.sync_copy(meta_hbm.at[b, 5], kbv)
        iota = lax.iota(jnp.int32, _L)
        zf = jnp.zeros((_L,), jnp.float32)
        k = kbv[pl.ds(_N // _L, _L)].astype(jnp.int32)

        def rep(sc_i, carry):
            kfv = kbv[pl.ds(sc_i * _SCH, _L)]
            for t in range(_SCH):
                off = sc_i * (_SCH * _L) + t * _L
                kof_t = jnp.squeeze(
                    lax.slice(kfv, (t,), (t + 1,))).astype(jnp.int32)
                src = srcv[pl.ds(off, _L)].astype(jnp.int32)
                exv[pl.ds(kof_t, _L)] = (
                    _gather16(pxv[pl.ds(off, _L)], src) - clxv[pl.ds(kof_t, _L)])
                eyv[pl.ds(kof_t, _L)] = (
                    _gather16(pyv[pl.ds(off, _L)], src) - clyv[pl.ds(kof_t, _L)])
            return carry

        lax.fori_loop(0, _N // _L // _SCH, rep, 0)

        def pass2(i, carry):
            vacc, aacc = carry
            j0 = i * _L
            jv = j0 + iota
            p0 = cpfv[pl.ds(j0, _L)]
            p1 = cpfv[pl.ds(j0 + 1, _L)]
            p2 = cpfv[pl.ds(j0 + 2, _L)]
            ex0 = exv[pl.ds(j0, _L)]
            ex1 = exv[pl.ds(j0 + 1, _L)]
            ex2 = exv[pl.ds(j0 + 2, _L)]
            ey0 = eyv[pl.ds(j0, _L)]
            ey1 = eyv[pl.ds(j0 + 1, _L)]
            ey2 = eyv[pl.ds(j0 + 2, _L)]
            w1 = mfv[pl.ds(j0 + 1, _L)]
            w2 = mfv[pl.ds(j0 + 2, _L)]
            g1 = _FPS / (p1 - p0)
            g2 = _FPS / (p2 - p1)
            dvx = (ex1 - ex0) * g1
            dvy = (ey1 - ey0) * g1
            vsq = dvx * dvx + dvy * dvy
            vacc = vacc + jnp.where(jv < k - 1, vsq * w1, 0.0)
            dax = ((ex2 - ex1) * g2 - dvx) * g2
            day = ((ey2 - ey1) * g2 - dvy) * g2
            asq = dax * dax + day * day
            aacc = aacc + jnp.where(jv < k - 2, asq * w2, 0.0)
            return vacc, aacc

        vacc, aacc = lax.fori_loop(0, _N // _L, pass2, (zf, zf))
        resv[0, :] = vacc
        resv[1, :] = aacc
        pltpu.sync_copy(resv, out_hbm.at[b])


_sc2_call = pl.kernel(
    _sc2_loss_body,
    out_type=jax.ShapeDtypeStruct((_B, 2, _L), jnp.float32),
    mesh=plsc.VectorSubcoreMesh(core_axis_name="c", subcore_axis_name="s"),
    scratch_types=[
        pltpu.VMEM((_N,), jnp.float32),
        pltpu.VMEM((_N,), jnp.float32),
        pltpu.VMEM((_CN,), jnp.float32),
        pltpu.VMEM((_CN,), jnp.float32),
        pltpu.VMEM((_CN,), jnp.float32),
        pltpu.VMEM((_CN,), jnp.float32),
        pltpu.VMEM((_CN,), jnp.float32),
        pltpu.VMEM((_CN,), jnp.float32),
        pltpu.VMEM((_CN,), jnp.float32),
        pltpu.VMEM((_CN,), jnp.float32),
        pltpu.VMEM((2, _L), jnp.float32),
    ],
)


def kernel(heatmapsx, heatmapsy, labels):
    lx = labels[:, :, 0]
    ly = labels[:, :, 1]
    meta, dens = _sc1_call(lx, ly)
    coords = _argmax_call(heatmapsx.reshape(_GR, _RPS, _S),
                          heatmapsy.reshape(_GR, _RPS, _S))
    px = coords[:, 0, :].reshape(_B, _N)
    py = coords[:, 1, :].reshape(_B, _N)
    partials = _sc2_call(px, py, meta)
    vel_num = jnp.sum(partials[:, 0, :])
    acc_num = jnp.sum(partials[:, 1, :])
    den1 = jnp.sum(dens[:, 0, :])
    den2 = jnp.sum(dens[:, 1, :])
    total = vel_num / den1 + 0.1 * acc_num / den2
    return _LOSS_SCALE * total


# speedup vs baseline: 1.0201x; 1.0201x over previous
import jax
import jax.numpy as jnp
from jax import lax
from jax.experimental import pallas as pl
from jax.experimental.pallas import tpu as pltpu
from jax.experimental.pallas import tpu_sc as plsc

_FPS = 25.0
_LOSS_SCALE = 1e-07
_B, _N, _S = 16, 2048, 512
_CN = _N + 16
_L = 16


_GR = 16
_RPS = _B * _N // _GR


def _argmax_body(hx_ref, hy_ref, out_ref):
    iot = lax.broadcasted_iota(jnp.int32, (_RPS, _S), 1)
    x = hx_ref[0]
    mx = jnp.max(x, axis=-1, keepdims=True)
    ax = jnp.min(jnp.where(x == mx, iot, _S), axis=-1)
    y = hy_ref[0]
    my = jnp.max(y, axis=-1, keepdims=True)
    ay = jnp.min(jnp.where(y == my, iot, _S), axis=-1)
    out_ref[0, 0, :] = ax.astype(jnp.float32)
    out_ref[0, 1, :] = ay.astype(jnp.float32)


_argmax_call = pl.pallas_call(
    _argmax_body,
    grid=(_GR,),
    in_specs=[
        pl.BlockSpec((1, _RPS, _S), lambda i: (i, 0, 0)),
        pl.BlockSpec((1, _RPS, _S), lambda i: (i, 0, 0)),
    ],
    out_specs=pl.BlockSpec((1, 2, _RPS), lambda i: (i, 0, 0)),
    out_shape=jax.ShapeDtypeStruct((_GR, 2, _RPS), jnp.float32),
)


_GATHER_DNUMS = lax.GatherDimensionNumbers(
    offset_dims=(), collapsed_slice_dims=(0,), start_index_map=(0,))


def _gather16(x, idx):
    return lax.gather(x, idx[:, None], _GATHER_DNUMS, (1,),
                      mode=lax.GatherScatterMode.PROMISE_IN_BOUNDS)


def _sc_loss_body(px_hbm, py_hbm, lx_hbm, ly_hbm, out_hbm,
                  pxv, pyv, lxv, lyv, mfv, cpv, cpxv, cpyv, resv):
    c = lax.axis_index("c")
    s = lax.axis_index("s")
    wid = s * 2 + c

    @pl.when(wid < _B)
    def _():
        b = wid
        pltpu.sync_copy(px_hbm.at[b], pxv)
        pltpu.sync_copy(py_hbm.at[b], pyv)
        pltpu.sync_copy(lx_hbm.at[b], lxv)
        pltpu.sync_copy(ly_hbm.at[b], lyv)
        iota = lax.iota(jnp.int32, _L)
        zf = jnp.zeros((_L,), jnp.float32)
        zi = jnp.zeros((_L,), jnp.int32)

        def pass1(i, carry):
            kofs, kvec, den1, den2 = carry
            off = i * _L
            lx = lxv[pl.ds(off, _L)]
            ly = lyv[pl.ds(off, _L)]
            m = jnp.logical_and(lx != 0.0, ly != 0.0)
            mi = jnp.where(m, 1, 0)
            mf = jnp.where(m, 1.0, 0.0)
            incl = mi
            for d in (1, 2, 4, 8):
                sh = _gather16(incl, jnp.maximum(iota - d, 0))
                incl = incl + jnp.where(iota >= d, sh, 0)
            src = zi
            for w in (8, 4, 2, 1):
                probe = _gather16(incl, src + (w - 1))
                src = src + jnp.where(probe < iota + 1, w, 0)
            cpv[pl.ds(kofs, _L)] = off + src
            cpxv[pl.ds(kofs, _L)] = _gather16(pxv[pl.ds(off, _L)] - lx, src)
            cpyv[pl.ds(kofs, _L)] = _gather16(pyv[pl.ds(off, _L)] - ly, src)
            mfv[pl.ds(off, _L)] = mf
            pos = off + iota
            den1 = den1 + jnp.where(pos >= 1, mf, 0.0)
            den2 = den2 + jnp.where(pos >= 2, mf, 0.0)
            cnt = jnp.squeeze(lax.slice(incl, (_L - 1,), (_L,)))
            kvec = kvec + _gather16(incl, jnp.full((_L,), _L - 1, jnp.int32))
            return kofs + cnt, kvec, den1, den2

        kofs, k, den1, den2 = lax.fori_loop(
            0, _N // _L, pass1, (jnp.int32(0), zi, zf, zf))

        def pass2(i, carry):
            vacc, aacc = carry
            j0 = i * _L
            jv = j0 + iota
            p0 = cpv[pl.ds(j0, _L)]
            p1 = cpv[pl.ds(j0 + 1, _L)]
            p2 = cpv[pl.ds(j0 + 2, _L)]
            ex0 = cpxv[pl.ds(j0, _L)]
            ex1 = cpxv[pl.ds(j0 + 1, _L)]
            ex2 = cpxv[pl.ds(j0 + 2, _L)]
            ey0 = cpyv[pl.ds(j0, _L)]
            ey1 = cpyv[pl.ds(j0 + 1, _L)]
            ey2 = cpyv[pl.ds(j0 + 2, _L)]
            w1 = mfv[pl.ds(j0 + 1, _L)]
            w2 = mfv[pl.ds(j0 + 2, _L)]
            g1 = _FPS / (p1 - p0).astype(jnp.float32)
            g2 = _FPS / (p2 - p1).astype(jnp.float32)
            dvx = (ex1 - ex0) * g1
            dvy = (ey1 - ey0) * g1
            vsq = dvx * dvx + dvy * dvy
            vacc = vacc + jnp.where(jv < k - 1, vsq * w1, 0.0)
            dax = ((ex2 - ex1) * g2 - dvx) * g2
            day = ((ey2 - ey1) * g2 - dvy) * g2
            asq = dax * dax + day * day
            aacc = aacc + jnp.where(jv < k - 2, asq * w2, 0.0)
            return vacc, aacc

        vacc, aacc = lax.fori_loop(0, _N // _L, pass2, (zf, zf))
        resv[0, :] = vacc
        resv[1, :] = aacc
        resv[2, :] = den1
        resv[3, :] = den2
        pltpu.sync_copy(resv, out_hbm.at[b])


_sc_loss_call = pl.kernel(
    _sc_loss_body,
    out_type=jax.ShapeDtypeStruct((_B, 4, _L), jnp.float32),
    mesh=plsc.VectorSubcoreMesh(core_axis_name="c", subcore_axis_name="s"),
    scratch_types=[
        pltpu.VMEM((_N,), jnp.float32),
        pltpu.VMEM((_N,), jnp.float32),
        pltpu.VMEM((_N,), jnp.float32),
        pltpu.VMEM((_N,), jnp.float32),
        pltpu.VMEM((_CN,), jnp.float32),
        pltpu.VMEM((_CN,), jnp.int32),
        pltpu.VMEM((_CN,), jnp.float32),
        pltpu.VMEM((_CN,), jnp.float32),
        pltpu.VMEM((4, _L), jnp.float32),
    ],
)


def kernel(heatmapsx, heatmapsy, labels):
    coords = _argmax_call(heatmapsx.reshape(_GR, _RPS, _S),
                          heatmapsy.reshape(_GR, _RPS, _S))
    px = coords[:, 0, :].reshape(_B, _N)
    py = coords[:, 1, :].reshape(_B, _N)
    lx = labels[:, :, 0]
    ly = labels[:, :, 1]
    partials = _sc_loss_call(px, py, lx, ly)
    sums = jnp.sum(partials, axis=(0, 2))
    total = sums[0] / sums[2] + 0.1 * sums[1] / sums[3]
    return _LOSS_SCALE * total
